# baseline (speedup 1.0000x reference)
import functools

import jax
import jax.numpy as jnp
from jax import lax
from jax.experimental import pallas as pl
from jax.experimental.pallas import tpu as pltpu
from jax.experimental.pallas import tpu_sc as plsc

N_NODES = 10000
N_EDGES = 320000
CH = 128
NC = 2
NS = 16
NW = NC * NS
EDGES_PER_TILE = N_EDGES // NW
CHUNK = 80
SUPER = 25
NSTAGE = 5
NPAIR = (SUPER - 1) // 2
N_PAD = 10240
ROWS_PER_TILE = N_PAD // NS


def _zero_2d(ref, nrows, ncols):
    def body(i, _):
        r = i // (ncols // 16)
        c = i % (ncols // 16)
        ref[r, pl.ds(c * 16, 16)] = jnp.zeros((16,), jnp.float32)
        return 0
    lax.fori_loop(0, nrows * (ncols // 16), body, 0)


def _zero_1d(ref, n):
    def body(i, _):
        ref[pl.ds(i * 16, 16)] = jnp.zeros((16,), jnp.float32)
        return 0
    lax.fori_loop(0, n // 16, body, 0)


def _make_seg_sum(with_counts: bool):
    out_type = [jax.ShapeDtypeStruct((NC, N_PAD, CH), jnp.float32)]
    if with_counts:
        out_type.append(jax.ShapeDtypeStruct((NW, N_PAD), jnp.float32))

    scratch = [
        pltpu.VMEM((2 * SUPER, CHUNK), jnp.int32),
        pltpu.VMEM((N_PAD,), jnp.float32),
        pltpu.VMEM((CHUNK, CH), jnp.float32),
        pltpu.VMEM((CHUNK, CH), jnp.float32),
        pltpu.VMEM_SHARED((N_PAD, CH), jnp.float32),
        pltpu.SemaphoreType.DMA,
        pltpu.SemaphoreType.DMA,
        pltpu.SemaphoreType.DMA,
        pltpu.SemaphoreType.DMA,
    ]

    def body(x_hbm, edges_hbm, *rest):
        if with_counts:
            sum_out, cnt_out = rest[0], rest[1]
            rest = rest[2:]
        else:
            sum_out = rest[0]
            rest = rest[1:]
        (ev, cnt_local, rows_a, rows_b, acc,
         sem_ga, sem_gb, sem_sa, sem_sb) = rest

        cid = lax.axis_index("c")
        sid = lax.axis_index("s")
        wid = sid * NC + cid

        if with_counts:
            _zero_1d(cnt_local, N_PAD)

        _zero_2d(rows_a, CHUNK, CH)
        base = sid * ROWS_PER_TILE
        nfull = ROWS_PER_TILE // CHUNK
        for k in range(nfull):
            pltpu.async_copy(rows_a, acc.at[pl.ds(base + k * CHUNK, CHUNK)],
                             sem_ga)
        for k in range(nfull):
            pltpu.make_async_copy(
                rows_a, acc.at[pl.ds(base + k * CHUNK, CHUNK)], sem_ga).wait()
        plsc.subcore_barrier()

        ones = jnp.ones((16,), jnp.float32)

        def gstart(j, buf, sem):
            pltpu.async_copy(x_hbm.at[ev.at[j]], buf, sem)

        def gwait(buf, sem):
            pltpu.make_async_copy(x_hbm.at[ev.at[0]], buf, sem).wait()

        def sstart(j, buf, sem):
            pltpu.async_copy(buf, acc.at[ev.at[SUPER + j]], sem, add=True)

        def swait(buf, sem):
            pltpu.make_async_copy(buf, acc.at[ev.at[SUPER]], sem).wait()

        def counts(j):
            if with_counts:
                for k in range(CHUNK // 16):
                    d = ev[SUPER + j, pl.ds(k * 16, 16)]
                    plsc.addupdate_scatter(cnt_local, [d], ones)

        for g in range(NSTAGE):
            pltpu.sync_copy(edges_hbm.at[wid, g], ev)
            gstart(0, rows_a, sem_ga)

            def pair(p, carry):
                j = 2 * p
                gstart(j + 1, rows_b, sem_gb)
                gwait(rows_a, sem_ga)
                sstart(j, rows_a, sem_sa)
                counts(j)
                gwait(rows_b, sem_gb)
                sstart(j + 1, rows_b, sem_sb)
                counts(j + 1)
                swait(rows_a, sem_sa)

                @pl.when(p < NPAIR - 1)
                def _():
                    gstart(j + 2, rows_a, sem_ga)

                swait(rows_b, sem_sb)
                return carry

            lax.fori_loop(0, NPAIR, pair, 0)

            t = SUPER - 1
            gstart(t, rows_a, sem_ga)
            gwait(rows_a, sem_ga)
            sstart(t, rows_a, sem_sa)
            counts(t)
            swait(rows_a, sem_sa)

        plsc.subcore_barrier()

        pltpu.sync_copy(acc.at[pl.ds(sid * ROWS_PER_TILE, ROWS_PER_TILE)],
                        sum_out.at[cid, pl.ds(sid * ROWS_PER_TILE, ROWS_PER_TILE)])
        if with_counts:
            pltpu.sync_copy(cnt_local, cnt_out.at[wid])

    mesh = plsc.VectorSubcoreMesh(core_axis_name="c", subcore_axis_name="s")
    return pl.kernel(
        body, mesh=mesh, out_type=out_type, scratch_types=scratch,
        compiler_params=pltpu.CompilerParams(needs_layout_passes=False))


_seg_sum_counts = _make_seg_sum(True)
_seg_sum = _make_seg_sum(False)


ROW_BLK = 1024


def _mm_body(a_ref, w_ref, o_ref):
    o_ref[...] = jnp.dot(a_ref[...], w_ref[...],
                         preferred_element_type=jnp.float32)


def _mm(a, w):
    k = a.shape[1]
    m = w.shape[1]
    return pl.pallas_call(
        _mm_body,
        grid=((N_NODES + ROW_BLK - 1) // ROW_BLK,),
        in_specs=[
            pl.BlockSpec((ROW_BLK, k), lambda i: (i, 0)),
            pl.BlockSpec((k, m), lambda i: (0, 0)),
        ],
        out_specs=pl.BlockSpec((ROW_BLK, m), lambda i: (i, 0)),
        out_shape=jax.ShapeDtypeStruct((N_NODES, m), jnp.float32),
    )(a, w)


def _tc1_body(s_ref, c_ref, xr_ref, w1l_ref, b1_ref, w2l_ref, h_ref, p_ref):
    s = s_ref[0] + s_ref[1]
    c = jnp.maximum(jnp.sum(c_ref[...], axis=0), 1.0)[:, None]
    mean = s / c
    z = (jnp.dot(mean, w1l_ref[...], preferred_element_type=jnp.float32)
         + xr_ref[...] + b1_ref[...])
    h = jnp.maximum(z, 0.0)
    h_ref[...] = h
    p_ref[...] = jnp.dot(h, w2l_ref[...], preferred_element_type=jnp.float32)


def _tc2_body(s_ref, c_ref, hr_ref, b2_ref, o_ref):
    s = s_ref[0] + s_ref[1]
    c = jnp.maximum(jnp.sum(c_ref[...], axis=0), 1.0)[:, None]
    o_ref[...] = s / c + hr_ref[...] + b2_ref[...]


def _tc1(s, cnt, xr, w1lt, b1, w2lt):
    grid = (N_PAD // ROW_BLK,)
    return pl.pallas_call(
        _tc1_body,
        grid=grid,
        in_specs=[
            pl.BlockSpec((NC, ROW_BLK, CH), lambda i: (0, i, 0)),
            pl.BlockSpec((NW, ROW_BLK), lambda i: (0, i)),
            pl.BlockSpec((ROW_BLK, 2 * CH), lambda i: (i, 0)),
            pl.BlockSpec((CH, 2 * CH), lambda i: (0, 0)),
            pl.BlockSpec((1, 2 * CH), lambda i: (0, 0)),
            pl.BlockSpec((2 * CH, CH), lambda i: (0, 0)),
        ],
        out_specs=[
            pl.BlockSpec((ROW_BLK, 2 * CH), lambda i: (i, 0)),
            pl.BlockSpec((ROW_BLK, CH), lambda i: (i, 0)),
        ],
        out_shape=[
            jax.ShapeDtypeStruct((N_NODES, 2 * CH), jnp.float32),
            jax.ShapeDtypeStruct((N_NODES, CH), jnp.float32),
        ],
    )(s, cnt, xr, w1lt, b1, w2lt)


def _tc2(s, cnt, hr, b2):
    grid = (N_PAD // ROW_BLK,)
    return pl.pallas_call(
        _tc2_body,
        grid=grid,
        in_specs=[
            pl.BlockSpec((NC, ROW_BLK, CH), lambda i: (0, i, 0)),
            pl.BlockSpec((NW, ROW_BLK), lambda i: (0, i)),
            pl.BlockSpec((ROW_BLK, CH), lambda i: (i, 0)),
            pl.BlockSpec((1, CH), lambda i: (0, 0)),
        ],
        out_specs=pl.BlockSpec((ROW_BLK, CH), lambda i: (i, 0)),
        out_shape=jax.ShapeDtypeStruct((N_NODES, CH), jnp.float32),
    )(s, cnt, hr, b2)


def kernel(x, edge_index, W1l, b1, W1r, W2l, b2, W2r):
    ei = edge_index.astype(jnp.int32).reshape(2, NW, NSTAGE, SUPER, CHUNK)
    edges = ei.transpose(1, 2, 0, 3, 4).reshape(NW, NSTAGE, 2 * SUPER, CHUNK)

    s1, cnt = _seg_sum_counts(x, edges)
    xr = _mm(x, W1r.T)
    h, p = _tc1(s1, cnt, xr, W1l.T, b1.reshape(1, -1), W2l.T)
    (s2,) = _seg_sum(p, edges)
    hr = _mm(h, W2r.T)
    out = _tc2(s2, cnt, hr, b2.reshape(1, -1))
    return out


# speedup vs baseline: 1.1798x; 1.1798x over previous
import functools

import jax
import jax.numpy as jnp
from jax import lax
from jax.experimental import pallas as pl
from jax.experimental.pallas import tpu as pltpu
from jax.experimental.pallas import tpu_sc as plsc

N_NODES = 10000
N_EDGES = 320000
CH = 128
NC = 2
NS = 16
NW = NC * NS
EDGES_PER_TILE = N_EDGES // NW
CHUNK = 80
SUPER = 25
NSTAGE = 5
NPAIR = (SUPER - 1) // 2
N_PAD = 10240
ROWS_PER_TILE = N_PAD // NS


def _zero_2d(ref, nrows, ncols):
    def body(i, _):
        r = i // (ncols // 16)
        c = i % (ncols // 16)
        ref[r, pl.ds(c * 16, 16)] = jnp.zeros((16,), jnp.float32)
        return 0
    lax.fori_loop(0, nrows * (ncols // 16), body, 0)


def _zero_1d(ref, n):
    def body(i, _):
        ref[pl.ds(i * 16, 16)] = jnp.zeros((16,), jnp.float32)
        return 0
    lax.fori_loop(0, n // 16, body, 0)


def _make_seg_sum(with_counts: bool):
    out_type = [jax.ShapeDtypeStruct((NC, N_PAD, CH), jnp.float32)]
    if with_counts:
        out_type.append(jax.ShapeDtypeStruct((NW, N_PAD), jnp.float32))

    scratch = [
        pltpu.VMEM((2 * SUPER, CHUNK), jnp.int32),
        pltpu.VMEM((N_PAD,), jnp.float32),
        pltpu.VMEM((CHUNK, CH), jnp.float32),
        pltpu.VMEM((CHUNK, CH), jnp.float32),
        pltpu.VMEM_SHARED((N_PAD, CH), jnp.float32),
        pltpu.SemaphoreType.DMA,
        pltpu.SemaphoreType.DMA,
        pltpu.SemaphoreType.DMA,
        pltpu.SemaphoreType.DMA,
    ]

    def body(x_hbm, edges_hbm, *rest):
        if with_counts:
            sum_out, cnt_out = rest[0], rest[1]
            rest = rest[2:]
        else:
            sum_out = rest[0]
            rest = rest[1:]
        (ev, cnt_local, rows_a, rows_b, acc,
         sem_ga, sem_gb, sem_sa, sem_sb) = rest

        cid = lax.axis_index("c")
        sid = lax.axis_index("s")
        wid = sid * NC + cid

        if with_counts:
            _zero_1d(cnt_local, N_PAD)

        _zero_2d(rows_a, CHUNK, CH)
        base = sid * ROWS_PER_TILE
        nfull = ROWS_PER_TILE // CHUNK
        for k in range(nfull):
            pltpu.async_copy(rows_a, acc.at[pl.ds(base + k * CHUNK, CHUNK)],
                             sem_ga)
        for k in range(nfull):
            pltpu.make_async_copy(
                rows_a, acc.at[pl.ds(base + k * CHUNK, CHUNK)], sem_ga).wait()
        plsc.subcore_barrier()

        ones = jnp.ones((16,), jnp.float32)

        def gstart(j, buf, sem):
            pltpu.async_copy(x_hbm.at[ev.at[j]], buf, sem)

        def gwait(buf, sem):
            pltpu.make_async_copy(x_hbm.at[ev.at[0]], buf, sem).wait()

        def sstart(j, buf, sem):
            pltpu.async_copy(buf, acc.at[ev.at[SUPER + j]], sem, add=True)

        def swait(buf, sem):
            pltpu.make_async_copy(buf, acc.at[ev.at[SUPER]], sem).wait()

        def counts(j):
            if with_counts:
                for k in range(CHUNK // 16):
                    d = ev[SUPER + j, pl.ds(k * 16, 16)]
                    plsc.addupdate_scatter(cnt_local, [d], ones)

        for g in range(NSTAGE):
            pltpu.sync_copy(edges_hbm.at[wid, g], ev)
            gstart(0, rows_a, sem_ga)

            def pair(p, carry):
                j = 2 * p
                gstart(j + 1, rows_b, sem_gb)
                gwait(rows_a, sem_ga)
                sstart(j, rows_a, sem_sa)
                counts(j)
                gwait(rows_b, sem_gb)
                swait(rows_a, sem_sa)

                @pl.when(p < NPAIR - 1)
                def _():
                    gstart(j + 2, rows_a, sem_ga)

                sstart(j + 1, rows_b, sem_sb)
                counts(j + 1)
                swait(rows_b, sem_sb)
                return carry

            lax.fori_loop(0, NPAIR, pair, 0)

            t = SUPER - 1
            gstart(t, rows_a, sem_ga)
            gwait(rows_a, sem_ga)
            sstart(t, rows_a, sem_sa)
            counts(t)
            swait(rows_a, sem_sa)

        plsc.subcore_barrier()

        pltpu.sync_copy(acc.at[pl.ds(sid * ROWS_PER_TILE, ROWS_PER_TILE)],
                        sum_out.at[cid, pl.ds(sid * ROWS_PER_TILE, ROWS_PER_TILE)])
        if with_counts:
            pltpu.sync_copy(cnt_local, cnt_out.at[wid])

    mesh = plsc.VectorSubcoreMesh(core_axis_name="c", subcore_axis_name="s")
    return pl.kernel(
        body, mesh=mesh, out_type=out_type, scratch_types=scratch,
        compiler_params=pltpu.CompilerParams(needs_layout_passes=False))


_seg_sum_counts = _make_seg_sum(True)
_seg_sum = _make_seg_sum(False)


ROW_BLK = 1024


def _mm_body(a_ref, w_ref, o_ref):
    o_ref[...] = jnp.dot(a_ref[...], w_ref[...],
                         preferred_element_type=jnp.float32)


def _mm(a, w):
    k = a.shape[1]
    m = w.shape[1]
    return pl.pallas_call(
        _mm_body,
        grid=((N_NODES + ROW_BLK - 1) // ROW_BLK,),
        in_specs=[
            pl.BlockSpec((ROW_BLK, k), lambda i: (i, 0)),
            pl.BlockSpec((k, m), lambda i: (0, 0)),
        ],
        out_specs=pl.BlockSpec((ROW_BLK, m), lambda i: (i, 0)),
        out_shape=jax.ShapeDtypeStruct((N_NODES, m), jnp.float32),
    )(a, w)


def _tc1_body(s_ref, c_ref, xr_ref, w1l_ref, b1_ref, w2l_ref, h_ref, p_ref):
    s = s_ref[0] + s_ref[1]
    c = jnp.maximum(jnp.sum(c_ref[...], axis=0), 1.0)[:, None]
    mean = s / c
    z = (jnp.dot(mean, w1l_ref[...], preferred_element_type=jnp.float32)
         + xr_ref[...] + b1_ref[...])
    h = jnp.maximum(z, 0.0)
    h_ref[...] = h
    p_ref[...] = jnp.dot(h, w2l_ref[...], preferred_element_type=jnp.float32)


def _tc2_body(s_ref, c_ref, hr_ref, b2_ref, o_ref):
    s = s_ref[0] + s_ref[1]
    c = jnp.maximum(jnp.sum(c_ref[...], axis=0), 1.0)[:, None]
    o_ref[...] = s / c + hr_ref[...] + b2_ref[...]


def _tc1(s, cnt, xr, w1lt, b1, w2lt):
    grid = (N_PAD // ROW_BLK,)
    return pl.pallas_call(
        _tc1_body,
        grid=grid,
        in_specs=[
            pl.BlockSpec((NC, ROW_BLK, CH), lambda i: (0, i, 0)),
            pl.BlockSpec((NW, ROW_BLK), lambda i: (0, i)),
            pl.BlockSpec((ROW_BLK, 2 * CH), lambda i: (i, 0)),
            pl.BlockSpec((CH, 2 * CH), lambda i: (0, 0)),
            pl.BlockSpec((1, 2 * CH), lambda i: (0, 0)),
            pl.BlockSpec((2 * CH, CH), lambda i: (0, 0)),
        ],
        out_specs=[
            pl.BlockSpec((ROW_BLK, 2 * CH), lambda i: (i, 0)),
            pl.BlockSpec((ROW_BLK, CH), lambda i: (i, 0)),
        ],
        out_shape=[
            jax.ShapeDtypeStruct((N_NODES, 2 * CH), jnp.float32),
            jax.ShapeDtypeStruct((N_NODES, CH), jnp.float32),
        ],
    )(s, cnt, xr, w1lt, b1, w2lt)


def _tc2(s, cnt, hr, b2):
    grid = (N_PAD // ROW_BLK,)
    return pl.pallas_call(
        _tc2_body,
        grid=grid,
        in_specs=[
            pl.BlockSpec((NC, ROW_BLK, CH), lambda i: (0, i, 0)),
            pl.BlockSpec((NW, ROW_BLK), lambda i: (0, i)),
            pl.BlockSpec((ROW_BLK, CH), lambda i: (i, 0)),
            pl.BlockSpec((1, CH), lambda i: (0, 0)),
        ],
        out_specs=pl.BlockSpec((ROW_BLK, CH), lambda i: (i, 0)),
        out_shape=jax.ShapeDtypeStruct((N_NODES, CH), jnp.float32),
    )(s, cnt, hr, b2)


def kernel(x, edge_index, W1l, b1, W1r, W2l, b2, W2r):
    ei = edge_index.astype(jnp.int32).reshape(2, NW, NSTAGE, SUPER, CHUNK)
    edges = ei.transpose(1, 2, 0, 3, 4).reshape(NW, NSTAGE, 2 * SUPER, CHUNK)

    s1, cnt = _seg_sum_counts(x, edges)
    xr = _mm(x, W1r.T)
    h, p = _tc1(s1, cnt, xr, W1l.T, b1.reshape(1, -1), W2l.T)
    (s2,) = _seg_sum(p, edges)
    hr = _mm(h, W2r.T)
    out = _tc2(s2, cnt, hr, b2.reshape(1, -1))
    return out


# speedup vs baseline: 1.3519x; 1.1459x over previous
import functools

import jax
import jax.numpy as jnp
from jax import lax
from jax.experimental import pallas as pl
from jax.experimental.pallas import tpu as pltpu
from jax.experimental.pallas import tpu_sc as plsc

N_NODES = 10000
N_EDGES = 320000
CH = 128
NC = 2
NS = 16
NW = NC * NS
EDGES_PER_TILE = N_EDGES // NW
CHUNK = 80
SUPER = 25
NSTAGE = 5
NTRI = SUPER // 3
N_PAD = 10240
ROWS_PER_TILE = N_PAD // NS


def _zero_2d(ref, nrows, ncols):
    def body(i, _):
        r = i // (ncols // 16)
        c = i % (ncols // 16)
        ref[r, pl.ds(c * 16, 16)] = jnp.zeros((16,), jnp.float32)
        return 0
    lax.fori_loop(0, nrows * (ncols // 16), body, 0)


def _zero_1d(ref, n):
    def body(i, _):
        ref[pl.ds(i * 16, 16)] = jnp.zeros((16,), jnp.float32)
        return 0
    lax.fori_loop(0, n // 16, body, 0)


def _make_seg_sum(with_counts: bool):
    out_type = [jax.ShapeDtypeStruct((NC, N_PAD, CH), jnp.float32)]
    if with_counts:
        out_type.append(jax.ShapeDtypeStruct((NW, N_PAD), jnp.float32))

    scratch = [
        pltpu.VMEM((2 * SUPER, CHUNK), jnp.int32),
        pltpu.VMEM((N_PAD,), jnp.float32),
        pltpu.VMEM((CHUNK, CH), jnp.float32),
        pltpu.VMEM((CHUNK, CH), jnp.float32),
        pltpu.VMEM((CHUNK, CH), jnp.float32),
        pltpu.VMEM_SHARED((N_PAD, CH), jnp.float32),
        pltpu.SemaphoreType.DMA,
        pltpu.SemaphoreType.DMA,
        pltpu.SemaphoreType.DMA,
        pltpu.SemaphoreType.DMA,
    ]

    def body(x_hbm, edges_hbm, *rest):
        if with_counts:
            sum_out, cnt_out = rest[0], rest[1]
            rest = rest[2:]
        else:
            sum_out = rest[0]
            rest = rest[1:]
        (ev, cnt_local, rows_a, rows_b, rows_c, acc,
         sem_ga, sem_gb, sem_gc, sem_ss) = rest

        cid = lax.axis_index("c")
        sid = lax.axis_index("s")
        wid = sid * NC + cid

        if with_counts:
            _zero_1d(cnt_local, N_PAD)

        _zero_2d(rows_a, CHUNK, CH)
        base = sid * ROWS_PER_TILE
        nfull = ROWS_PER_TILE // CHUNK
        for k in range(nfull):
            pltpu.async_copy(rows_a, acc.at[pl.ds(base + k * CHUNK, CHUNK)],
                             sem_ga)
        for k in range(nfull):
            pltpu.make_async_copy(
                rows_a, acc.at[pl.ds(base + k * CHUNK, CHUNK)], sem_ga).wait()
        plsc.subcore_barrier()

        ones = jnp.ones((16,), jnp.float32)

        def gstart(j, buf, sem):
            pltpu.async_copy(x_hbm.at[ev.at[j]], buf, sem)

        def gwait(buf, sem):
            pltpu.make_async_copy(x_hbm.at[ev.at[0]], buf, sem).wait()

        def sstart(j, buf, sem):
            pltpu.async_copy(buf, acc.at[ev.at[SUPER + j]], sem, add=True)

        def swait(buf, sem):
            pltpu.make_async_copy(buf, acc.at[ev.at[SUPER]], sem).wait()

        def counts(j):
            if with_counts:
                for k in range(CHUNK // 16):
                    d = ev[SUPER + j, pl.ds(k * 16, 16)]
                    plsc.addupdate_scatter(cnt_local, [d], ones)

        for g in range(NSTAGE):
            pltpu.sync_copy(edges_hbm.at[wid, g], ev)
            gstart(0, rows_a, sem_ga)
            gstart(1, rows_b, sem_gb)

            def tri(t, carry):
                j = 3 * t
                gstart(j + 2, rows_c, sem_gc)
                gwait(rows_a, sem_ga)
                sstart(j, rows_a, sem_ss)
                counts(j)
                gwait(rows_b, sem_gb)
                swait(rows_a, sem_ss)
                gstart(j + 3, rows_a, sem_ga)
                sstart(j + 1, rows_b, sem_ss)
                counts(j + 1)
                gwait(rows_c, sem_gc)
                swait(rows_b, sem_ss)

                @pl.when(t < NTRI - 1)
                def _():
                    gstart(j + 4, rows_b, sem_gb)

                sstart(j + 2, rows_c, sem_ss)
                counts(j + 2)
                swait(rows_c, sem_ss)
                return carry

            lax.fori_loop(0, NTRI, tri, 0)

            t24 = SUPER - 1
            gwait(rows_a, sem_ga)
            sstart(t24, rows_a, sem_ss)
            counts(t24)
            swait(rows_a, sem_ss)

        plsc.subcore_barrier()

        pltpu.sync_copy(acc.at[pl.ds(sid * ROWS_PER_TILE, ROWS_PER_TILE)],
                        sum_out.at[cid, pl.ds(sid * ROWS_PER_TILE, ROWS_PER_TILE)])
        if with_counts:
            pltpu.sync_copy(cnt_local, cnt_out.at[wid])

    mesh = plsc.VectorSubcoreMesh(core_axis_name="c", subcore_axis_name="s")
    return pl.kernel(
        body, mesh=mesh, out_type=out_type, scratch_types=scratch,
        compiler_params=pltpu.CompilerParams(needs_layout_passes=False))


_seg_sum_counts = _make_seg_sum(True)
_seg_sum = _make_seg_sum(False)


ROW_BLK = 1024


def _mm_body(a_ref, w_ref, o_ref):
    o_ref[...] = jnp.dot(a_ref[...], w_ref[...],
                         preferred_element_type=jnp.float32)


def _mm(a, w):
    k = a.shape[1]
    m = w.shape[1]
    return pl.pallas_call(
        _mm_body,
        grid=((N_NODES + ROW_BLK - 1) // ROW_BLK,),
        in_specs=[
            pl.BlockSpec((ROW_BLK, k), lambda i: (i, 0)),
            pl.BlockSpec((k, m), lambda i: (0, 0)),
        ],
        out_specs=pl.BlockSpec((ROW_BLK, m), lambda i: (i, 0)),
        out_shape=jax.ShapeDtypeStruct((N_NODES, m), jnp.float32),
    )(a, w)


def _tc1_body(s_ref, c_ref, xr_ref, w1l_ref, b1_ref, w2l_ref, h_ref, p_ref):
    s = s_ref[0] + s_ref[1]
    c = jnp.maximum(jnp.sum(c_ref[...], axis=0), 1.0)[:, None]
    mean = s / c
    z = (jnp.dot(mean, w1l_ref[...], preferred_element_type=jnp.float32)
         + xr_ref[...] + b1_ref[...])
    h = jnp.maximum(z, 0.0)
    h_ref[...] = h
    p_ref[...] = jnp.dot(h, w2l_ref[...], preferred_element_type=jnp.float32)


def _tc2_body(s_ref, c_ref, hr_ref, b2_ref, o_ref):
    s = s_ref[0] + s_ref[1]
    c = jnp.maximum(jnp.sum(c_ref[...], axis=0), 1.0)[:, None]
    o_ref[...] = s / c + hr_ref[...] + b2_ref[...]


def _tc1(s, cnt, xr, w1lt, b1, w2lt):
    grid = (N_PAD // ROW_BLK,)
    return pl.pallas_call(
        _tc1_body,
        grid=grid,
        in_specs=[
            pl.BlockSpec((NC, ROW_BLK, CH), lambda i: (0, i, 0)),
            pl.BlockSpec((NW, ROW_BLK), lambda i: (0, i)),
            pl.BlockSpec((ROW_BLK, 2 * CH), lambda i: (i, 0)),
            pl.BlockSpec((CH, 2 * CH), lambda i: (0, 0)),
            pl.BlockSpec((1, 2 * CH), lambda i: (0, 0)),
            pl.BlockSpec((2 * CH, CH), lambda i: (0, 0)),
        ],
        out_specs=[
            pl.BlockSpec((ROW_BLK, 2 * CH), lambda i: (i, 0)),
            pl.BlockSpec((ROW_BLK, CH), lambda i: (i, 0)),
        ],
        out_shape=[
            jax.ShapeDtypeStruct((N_NODES, 2 * CH), jnp.float32),
            jax.ShapeDtypeStruct((N_NODES, CH), jnp.float32),
        ],
    )(s, cnt, xr, w1lt, b1, w2lt)


def _tc2(s, cnt, hr, b2):
    grid = (N_PAD // ROW_BLK,)
    return pl.pallas_call(
        _tc2_body,
        grid=grid,
        in_specs=[
            pl.BlockSpec((NC, ROW_BLK, CH), lambda i: (0, i, 0)),
            pl.BlockSpec((NW, ROW_BLK), lambda i: (0, i)),
            pl.BlockSpec((ROW_BLK, CH), lambda i: (i, 0)),
            pl.BlockSpec((1, CH), lambda i: (0, 0)),
        ],
        out_specs=pl.BlockSpec((ROW_BLK, CH), lambda i: (i, 0)),
        out_shape=jax.ShapeDtypeStruct((N_NODES, CH), jnp.float32),
    )(s, cnt, hr, b2)


def kernel(x, edge_index, W1l, b1, W1r, W2l, b2, W2r):
    ei = edge_index.astype(jnp.int32).reshape(2, NW, NSTAGE, SUPER, CHUNK)
    edges = ei.transpose(1, 2, 0, 3, 4).reshape(NW, NSTAGE, 2 * SUPER, CHUNK)

    s1, cnt = _seg_sum_counts(x, edges)
    xr = _mm(x, W1r.T)
    h, p = _tc1(s1, cnt, xr, W1l.T, b1.reshape(1, -1), W2l.T)
    (s2,) = _seg_sum(p, edges)
    hr = _mm(h, W2r.T)
    out = _tc2(s2, cnt, hr, b2.reshape(1, -1))
    return out
